# baseline (speedup 1.0000x reference)
import functools

import jax
import jax.numpy as jnp
from jax.experimental import pallas as pl
from jax.experimental.pallas import tpu as pltpu

_LANES = 128
_POST = 300
_PRE = 6000
_MIN_SIZE = 16.0
_OUT_ROWS = 304


def _body(s_ref, d_ref, anc_ref, info_ref, prm_ref, out_ref, act_ref):
    rows = s_ref.shape[2]
    s0 = s_ref[0, 0]
    s1 = s_ref[0, 1]
    d0 = d_ref[0, 0]
    d1 = d_ref[0, 1]
    d2 = d_ref[0, 2]
    d3 = d_ref[0, 3]
    a0 = anc_ref[0]
    a1 = anc_ref[1]
    a2 = anc_ref[2]
    a3 = anc_ref[3]
    info_h = info_ref[0, 0, 0]
    info_w = info_ref[0, 0, 1]
    info_s2 = info_ref[0, 0, 2]
    info_s3 = info_ref[0, 0, 3]
    pre_nms = prm_ref[0, 0]
    post_nms = prm_ref[0, 1]
    thr = prm_ref[0, 2]

    m = jnp.maximum(s0, s1)
    e0 = jnp.exp(s0 - m)
    e1 = jnp.exp(s1 - m)
    sc = e1 / (e0 + e1)

    widths = a2 - a0 + 1.0
    heights = a3 - a1 + 1.0
    ctr_x = a0 + 0.5 * widths
    ctr_y = a1 + 0.5 * heights
    pcx = d0 * widths + ctr_x
    pcy = d1 * heights + ctr_y
    pw = jnp.exp(d2) * widths
    ph = jnp.exp(d3) * heights
    x1 = jnp.clip(pcx - 0.5 * pw, 0.0, info_w - 1.0)
    y1 = jnp.clip(pcy - 0.5 * ph, 0.0, info_h - 1.0)
    x2 = jnp.clip(pcx + 0.5 * pw, 0.0, info_w - 1.0)
    y2 = jnp.clip(pcy + 0.5 * ph, 0.0, info_h - 1.0)
    ws = x2 - x1 + 1.0
    hs = y2 - y1 + 1.0
    keep = (ws >= _MIN_SIZE * info_s3) & (hs >= _MIN_SIZE * info_s2)
    areas = (x2 - x1 + 1.0) * (y2 - y1 + 1.0)

    sbits = jax.lax.bitcast_convert_type(sc, jnp.int32)
    neg = jnp.int32(-2147483648)
    u = jnp.where(keep, sbits, neg)
    kk = jnp.minimum(pre_nms, jnp.float32(_PRE))

    def _tbit(i, t):
        cand = t | jax.lax.shift_left(jnp.int32(1), 30 - i)
        cnt = jnp.sum((u >= cand).astype(jnp.float32), axis=(0, 1), keepdims=True)
        return jnp.where(cnt >= kk, cand, t)

    tval = jax.lax.fori_loop(0, 31, _tbit, jnp.zeros((1, 1), jnp.int32))
    c_above = jnp.sum((u > tval).astype(jnp.float32), axis=(0, 1), keepdims=True)
    rem = kk - c_above

    ri = jax.lax.broadcasted_iota(jnp.int32, (rows, _LANES), 0)
    li = jax.lax.broadcasted_iota(jnp.int32, (rows, _LANES), 1)
    jidx = ri * _LANES + li
    eq = u == tval

    def _jbit(i, t):
        cand = t | jax.lax.shift_left(jnp.int32(1), 14 - i)
        cnt = jnp.sum((eq & (jidx < cand)).astype(jnp.float32), axis=(0, 1), keepdims=True)
        return jnp.where(cnt < rem, cand, t)

    jcut = jax.lax.fori_loop(0, 15, _jbit, jnp.zeros((1, 1), jnp.int32))
    cand_mask = (u > tval) | (eq & (jidx <= jcut))

    idf = pl.program_id(0).astype(jnp.float32)
    lane_row = jax.lax.broadcasted_iota(jnp.int32, (1, _LANES), 1)
    big = jnp.int32(1 << 30)

    act_ref[...] = jnp.where(cand_mask, 1.0, 0.0)

    def _pick(t, carry):
        act = act_ref[...] > 0.0
        sa = jnp.where(act, sc, -1.0)
        mx = jnp.max(sa, axis=(0, 1), keepdims=True)
        selj = jnp.min(jnp.where(act & (sa == mx), jidx, big),
                       axis=(0, 1), keepdims=True)
        ohf = (jidx == selj).astype(jnp.float32)
        bx1 = jnp.sum(ohf * x1, axis=(0, 1), keepdims=True)
        by1 = jnp.sum(ohf * y1, axis=(0, 1), keepdims=True)
        bx2 = jnp.sum(ohf * x2, axis=(0, 1), keepdims=True)
        by2 = jnp.sum(ohf * y2, axis=(0, 1), keepdims=True)
        ba = jnp.sum(ohf * areas, axis=(0, 1), keepdims=True)
        xx1 = jnp.maximum(bx1, x1)
        yy1 = jnp.maximum(by1, y1)
        xx2 = jnp.minimum(bx2, x2)
        yy2 = jnp.minimum(by2, y2)
        iw = jnp.maximum(0.0, xx2 - xx1 + 1.0)
        ih = jnp.maximum(0.0, yy2 - yy1 + 1.0)
        inter = iw * ih
        iou = inter / (ba + areas - inter)
        act_ref[...] = jnp.where(act & jnp.logical_not(iou > thr), 1.0, 0.0)
        wf = (t.astype(jnp.float32) < post_nms).astype(jnp.float32)
        row = jnp.where(lane_row == 0, idf, 0.0)
        row = row + jnp.where(lane_row == 1, bx1 * wf, 0.0)
        row = row + jnp.where(lane_row == 2, by1 * wf, 0.0)
        row = row + jnp.where(lane_row == 3, bx2 * wf, 0.0)
        row = row + jnp.where(lane_row == 4, by2 * wf, 0.0)
        out_ref[0, pl.ds(t, 1), :] = row
        return carry

    jax.lax.fori_loop(0, _POST, _pick, jnp.int32(0))


@jax.jit
def _run(scores, bbox_deltas, anchors, im_info, pre_nms, post_nms, nms_thresh):
    b, c2, h, w = scores.shape
    a = c2 // 2
    n = a * h * w
    rows = n // _LANES
    sc2 = jnp.transpose(scores, (0, 2, 3, 1)).reshape(b, n, 2)
    s_in = jnp.transpose(sc2, (0, 2, 1)).reshape(b, 2, rows, _LANES)
    dl4 = jnp.transpose(bbox_deltas, (0, 2, 3, 1)).reshape(b, n, 4)
    d_in = jnp.transpose(dl4, (0, 2, 1)).reshape(b, 4, rows, _LANES)
    anc = jnp.transpose(anchors.reshape(n, 4), (1, 0)).reshape(4, rows, _LANES)
    prm = jnp.stack([
        jnp.asarray(pre_nms).astype(jnp.float32),
        jnp.asarray(post_nms).astype(jnp.float32),
        jnp.asarray(nms_thresh).astype(jnp.float32),
        jnp.float32(0.0),
    ]).reshape(1, 4)
    out = pl.pallas_call(
        _body,
        grid=(b,),
        in_specs=[
            pl.BlockSpec((1, 2, rows, _LANES), lambda i: (i, 0, 0, 0)),
            pl.BlockSpec((1, 4, rows, _LANES), lambda i: (i, 0, 0, 0)),
            pl.BlockSpec((4, rows, _LANES), lambda i: (0, 0, 0)),
            pl.BlockSpec((1, 1, 4), lambda i: (i, 0, 0), memory_space=pltpu.SMEM),
            pl.BlockSpec((1, 4), lambda i: (0, 0), memory_space=pltpu.SMEM),
        ],
        out_specs=pl.BlockSpec((1, _OUT_ROWS, _LANES), lambda i: (i, 0, 0)),
        out_shape=jax.ShapeDtypeStruct((b, _OUT_ROWS, _LANES), jnp.float32),
        scratch_shapes=[pltpu.VMEM((rows, _LANES), jnp.float32)],
        compiler_params=pltpu.CompilerParams(
            dimension_semantics=("parallel",)),
    )(s_in, d_in, anc, im_info.reshape(b, 1, 4), prm)
    return out[:, :_POST, :5]


def kernel(scores, bbox_deltas, anchors, num_anchors, im_info, pre_nms, post_nms, nms_thresh):
    del num_anchors
    return _run(scores, bbox_deltas, anchors, im_info, pre_nms, post_nms, nms_thresh)


# speedup vs baseline: 2.6825x; 2.6825x over previous
import jax
import jax.numpy as jnp
from jax.experimental import pallas as pl
from jax.experimental.pallas import tpu as pltpu

_LANES = 128
_POST = 300
_PRE = 6000
_MIN_SIZE = 16.0
_OUT_ROWS = 304


def _body(s_ref, d_ref, anc_ref, info_ref, prm_ref, out_ref, act_ref):
    batch = s_ref.shape[0]
    rows = s_ref.shape[2]
    shape3 = (batch, rows, _LANES)
    s0 = s_ref[:, 0]
    s1 = s_ref[:, 1]
    d0 = d_ref[:, 0]
    d1 = d_ref[:, 1]
    d2 = d_ref[:, 2]
    d3 = d_ref[:, 3]
    a0 = anc_ref[0][None]
    a1 = anc_ref[1][None]
    a2 = anc_ref[2][None]
    a3 = anc_ref[3][None]
    info = info_ref[...]
    info_h = jax.lax.slice(info, (0, 0), (batch, 1))[:, :, None]
    info_w = jax.lax.slice(info, (0, 1), (batch, 2))[:, :, None]
    info_s2 = jax.lax.slice(info, (0, 2), (batch, 3))[:, :, None]
    info_s3 = jax.lax.slice(info, (0, 3), (batch, 4))[:, :, None]
    pre_nms = prm_ref[0, 0]
    post_nms = prm_ref[0, 1]
    thr = prm_ref[0, 2]

    m = jnp.maximum(s0, s1)
    e0 = jnp.exp(s0 - m)
    e1 = jnp.exp(s1 - m)
    sc = e1 / (e0 + e1)

    widths = a2 - a0 + 1.0
    heights = a3 - a1 + 1.0
    ctr_x = a0 + 0.5 * widths
    ctr_y = a1 + 0.5 * heights
    pcx = d0 * widths + ctr_x
    pcy = d1 * heights + ctr_y
    pw = jnp.exp(d2) * widths
    ph = jnp.exp(d3) * heights
    x1 = jnp.clip(pcx - 0.5 * pw, 0.0, info_w - 1.0)
    y1 = jnp.clip(pcy - 0.5 * ph, 0.0, info_h - 1.0)
    x2 = jnp.clip(pcx + 0.5 * pw, 0.0, info_w - 1.0)
    y2 = jnp.clip(pcy + 0.5 * ph, 0.0, info_h - 1.0)
    ws = x2 - x1 + 1.0
    hs = y2 - y1 + 1.0
    keep = (ws >= _MIN_SIZE * info_s3) & (hs >= _MIN_SIZE * info_s2)
    areas = (x2 - x1 + 1.0) * (y2 - y1 + 1.0)

    sbits = jax.lax.bitcast_convert_type(sc, jnp.int32)
    neg = jnp.int32(-2147483648)
    u = jnp.where(keep, sbits, neg)
    kk = jnp.minimum(pre_nms, jnp.float32(_PRE))

    def _tbit(i, t):
        cand = t | jax.lax.shift_left(jnp.int32(1), 30 - i)
        cnt = jnp.sum((u >= cand).astype(jnp.float32), axis=(1, 2),
                      keepdims=True)
        return jnp.where(cnt >= kk, cand, t)

    tval = jax.lax.fori_loop(0, 31, _tbit, jnp.zeros((batch, 1, 1), jnp.int32))
    c_above = jnp.sum((u > tval).astype(jnp.float32), axis=(1, 2),
                      keepdims=True)
    rem = kk - c_above

    ri = jax.lax.broadcasted_iota(jnp.int32, shape3, 1)
    li = jax.lax.broadcasted_iota(jnp.int32, shape3, 2)
    jidx = ri * _LANES + li
    eq = u == tval

    def _jbit(i, t):
        cand = t | jax.lax.shift_left(jnp.int32(1), 14 - i)
        cnt = jnp.sum((eq & (jidx < cand)).astype(jnp.float32), axis=(1, 2),
                      keepdims=True)
        return jnp.where(cnt < rem, cand, t)

    jcut = jax.lax.fori_loop(0, 15, _jbit, jnp.zeros((batch, 1, 1), jnp.int32))
    cand_mask = (u > tval) | (eq & (jidx <= jcut))

    ids = jax.lax.broadcasted_iota(
        jnp.int32, (batch, 1, _LANES), 0).astype(jnp.float32)
    lane_row = jax.lax.broadcasted_iota(jnp.int32, (batch, 1, _LANES), 2)
    big = jnp.int32(1 << 30)
    act_ref[...] = jnp.where(cand_mask, sc, -1.0)

    def _pick(t, carry):
        sa = act_ref[...]
        mx = jnp.max(sa, axis=(1, 2), keepdims=True)
        selj = jnp.min(jnp.where(sa == mx, jidx, big),
                       axis=(1, 2), keepdims=True)
        ohf = (jidx == selj).astype(jnp.float32)
        bx1 = jnp.sum(ohf * x1, axis=(1, 2), keepdims=True)
        by1 = jnp.sum(ohf * y1, axis=(1, 2), keepdims=True)
        bx2 = jnp.sum(ohf * x2, axis=(1, 2), keepdims=True)
        by2 = jnp.sum(ohf * y2, axis=(1, 2), keepdims=True)
        ba = jnp.sum(ohf * areas, axis=(1, 2), keepdims=True)
        xx1 = jnp.maximum(bx1, x1)
        yy1 = jnp.maximum(by1, y1)
        xx2 = jnp.minimum(bx2, x2)
        yy2 = jnp.minimum(by2, y2)
        iw = jnp.maximum(0.0, xx2 - xx1 + 1.0)
        ih = jnp.maximum(0.0, yy2 - yy1 + 1.0)
        inter = iw * ih
        iou = inter / (ba + areas - inter)
        act_ref[...] = jnp.where(iou > thr, -1.0, sa)
        wf = (t.astype(jnp.float32) < post_nms).astype(jnp.float32)
        vld = (mx > -0.5).astype(jnp.float32) * wf
        row = jnp.where(lane_row == 0, ids, 0.0)
        row = row + jnp.where(lane_row == 1, bx1 * vld, 0.0)
        row = row + jnp.where(lane_row == 2, by1 * vld, 0.0)
        row = row + jnp.where(lane_row == 3, bx2 * vld, 0.0)
        row = row + jnp.where(lane_row == 4, by2 * vld, 0.0)
        out_ref[:, pl.ds(t, 1), :] = row
        return carry

    jax.lax.fori_loop(0, _POST, _pick, jnp.int32(0))


@jax.jit
def _run(scores, bbox_deltas, anchors, im_info, pre_nms, post_nms, nms_thresh):
    b, c2, h, w = scores.shape
    a = c2 // 2
    n = a * h * w
    rows = n // _LANES
    sc2 = jnp.transpose(scores, (0, 2, 3, 1)).reshape(b, n, 2)
    s_in = jnp.transpose(sc2, (0, 2, 1)).reshape(b, 2, rows, _LANES)
    dl4 = jnp.transpose(bbox_deltas, (0, 2, 3, 1)).reshape(b, n, 4)
    d_in = jnp.transpose(dl4, (0, 2, 1)).reshape(b, 4, rows, _LANES)
    anc = jnp.transpose(anchors.reshape(n, 4), (1, 0)).reshape(4, rows, _LANES)
    prm = jnp.stack([
        jnp.asarray(pre_nms).astype(jnp.float32),
        jnp.asarray(post_nms).astype(jnp.float32),
        jnp.asarray(nms_thresh).astype(jnp.float32),
        jnp.float32(0.0),
    ]).reshape(1, 4)
    out = pl.pallas_call(
        _body,
        in_specs=[
            pl.BlockSpec(memory_space=pltpu.VMEM),
            pl.BlockSpec(memory_space=pltpu.VMEM),
            pl.BlockSpec(memory_space=pltpu.VMEM),
            pl.BlockSpec(memory_space=pltpu.VMEM),
            pl.BlockSpec(memory_space=pltpu.SMEM),
        ],
        out_specs=pl.BlockSpec(memory_space=pltpu.VMEM),
        out_shape=jax.ShapeDtypeStruct((b, _OUT_ROWS, _LANES), jnp.float32),
        scratch_shapes=[pltpu.VMEM((b, rows, _LANES), jnp.float32)],
    )(s_in, d_in, anc, im_info, prm)
    return out[:, :_POST, :5]


def kernel(scores, bbox_deltas, anchors, num_anchors, im_info, pre_nms, post_nms, nms_thresh):
    del num_anchors
    return _run(scores, bbox_deltas, anchors, im_info, pre_nms, post_nms, nms_thresh)
